# baseline (speedup 1.0000x reference)
import functools

import jax
import jax.numpy as jnp
from jax import lax
from jax.experimental import pallas as pl
from jax.experimental.pallas import tpu as pltpu
from jax.experimental.pallas import tpu_sc as plsc

VOCAB_P1 = 100001
EMBED_DIM = 32
BATCH = 16384

_NUM_CORES = 2
_NUM_SUBCORES = 16
_NUM_WORKERS = _NUM_CORES * _NUM_SUBCORES
_B_PER_W = BATCH // _NUM_WORKERS

_mesh = plsc.VectorSubcoreMesh(core_axis_name="c", subcore_axis_name="s")


@functools.partial(
    pl.kernel,
    mesh=_mesh,
    out_type=jax.ShapeDtypeStruct((BATCH, EMBED_DIM), jnp.float32),
    scratch_types=[
        pltpu.VMEM((_B_PER_W,), jnp.int32),
        pltpu.VMEM((_B_PER_W, EMBED_DIM), jnp.float32),
        pltpu.SemaphoreType.DMA,
    ],
    compiler_params=pltpu.CompilerParams(use_tc_tiling_on_sc=False),
)
def _gather_rows(table_hbm, idx_hbm, out_hbm, idx_v, rows_v, sem):
    wid = lax.axis_index("s") * _NUM_CORES + lax.axis_index("c")
    base = wid * _B_PER_W
    pltpu.sync_copy(idx_hbm.at[pl.ds(base, _B_PER_W)], idx_v)
    pltpu.async_copy(table_hbm.at[idx_v], rows_v, sem).wait()
    pltpu.sync_copy(rows_v, out_hbm.at[pl.ds(base, _B_PER_W)])


@jax.jit
def kernel(customer_id, user_embedding_table):
    return _gather_rows(user_embedding_table, customer_id)


# speedup vs baseline: 2.2940x; 2.2940x over previous
import functools

import jax
import jax.numpy as jnp
from jax import lax
from jax.experimental import pallas as pl
from jax.experimental.pallas import tpu as pltpu
from jax.experimental.pallas import tpu_sc as plsc

VOCAB_P1 = 100001
EMBED_DIM = 32
BATCH = 16384
_LANES = 16

_NUM_CORES = 2
_HALF = BATCH // 2

_mesh = plsc.VectorSubcoreMesh(core_axis_name="c", subcore_axis_name="s")


@functools.partial(
    pl.kernel,
    mesh=_mesh,
    out_type=jax.ShapeDtypeStruct((EMBED_DIM, BATCH), jnp.float32),
    scratch_types=[
        pltpu.VMEM((VOCAB_P1,), jnp.float32),
        pltpu.VMEM((BATCH,), jnp.int32),
        pltpu.VMEM((_HALF,), jnp.float32),
    ],
    compiler_params=pltpu.CompilerParams(needs_layout_passes=False),
)
def _gather_feature_major(table_t_hbm, idx_hbm, out_t_hbm, row_v, idx_v, out_v):
    dim = lax.axis_index("s") * _NUM_CORES + lax.axis_index("c")
    pltpu.sync_copy(table_t_hbm.at[dim], row_v)
    pltpu.sync_copy(idx_hbm, idx_v)
    for half in range(2):
        def body(k, carry):
            iv = idx_v[pl.ds(half * _HALF + k * _LANES, _LANES)]
            out_v[pl.ds(k * _LANES, _LANES)] = plsc.load_gather(row_v, [iv])
            return carry
        lax.fori_loop(0, _HALF // _LANES, body, 0, unroll=8)
        pltpu.sync_copy(out_v, out_t_hbm.at[dim, pl.ds(half * _HALF, _HALF)])


@jax.jit
def kernel(customer_id, user_embedding_table):
    out_t = _gather_feature_major(user_embedding_table.T, customer_id)
    return out_t.T
